# Initial kernel scaffold; baseline (speedup 1.0000x reference)
import jax
import jax.numpy as jnp
from jax.experimental import pallas as pl


def kernel(x, edge_index, edge_attr, batch, vec_data, block_emb, bond_emb, W1, b1, W2, b2, conv_root, conv_bias, W_ih, W_hh, b_ih, b_hh):
    raise NotImplementedError("write your pallas kernel here")


# speedup vs baseline: 16.4801x; 16.4801x over previous
import functools

import jax
import jax.numpy as jnp
from jax import lax
from jax.experimental import pallas as pl
from jax.experimental.pallas import tpu as pltpu
from jax.experimental.pallas import tpu_sc as plsc

N = 10000
E = 160000
G = 512
NEMB = 32
NSTEM = 20
NBLK = 106
STEPS = 3

R = 1000
GRID = N // R

NC, NS = 2, 16
NW = NC * NS
CL = 128
CHUNKS = 40
EPW = CHUNKS * CL
EPAD = NW * EPW
QCOLS = NSTEM + 1
QSIZE = N * QCOLS
QS = 13128
QPAD = QS * NS
PSIZE = N * NSTEM


def _leaky(t):
    return jnp.where(t >= 0, t, 0.01 * t)


def _encode_body(x_ref, b_ref, vec_ref, blk_ref, w1_ref, b1_ref, w2_ref,
                 b2_ref, bond_ref, out_ref, p_ref):
    xb = x_ref[0, 0, :]
    oh_x = (xb[:, None] == lax.broadcasted_iota(jnp.int32, (R, NBLK), 1)
            ).astype(jnp.float32)
    hx = jnp.dot(oh_x, blk_ref[...], preferred_element_type=jnp.float32)
    bb = b_ref[0, 0, :]
    oh_b = (bb[:, None] == lax.broadcasted_iota(jnp.int32, (R, G), 1)
            ).astype(jnp.float32)
    bv = jnp.dot(oh_b, vec_ref[...], preferred_element_type=jnp.float32)
    cat = jnp.concatenate([hx, bv], axis=1)
    t = _leaky(jnp.dot(cat, w1_ref[...], preferred_element_type=jnp.float32)
               + b1_ref[...])
    out = jnp.dot(t, w2_ref[...], preferred_element_type=jnp.float32) + b2_ref[...]
    out_ref[...] = out
    p_ref[...] = lax.dot_general(out, bond_ref[...], (((1,), (1,)), ((), ())),
                                 preferred_element_type=jnp.float32)


def _step_body(o_ref, q_ref, bond_ref, cr_ref, cb_ref, wih_ref, whh_ref,
               bih_ref, bhh_ref, h_ref, p_ref):
    o = o_ref[...]
    q = q_ref[0] + q_ref[1]
    deg = jnp.maximum(q[:, NSTEM:NSTEM + 1], 1.0)
    aggr = jnp.dot(q[:, :NSTEM], bond_ref[...],
                   preferred_element_type=jnp.float32) / deg
    m = _leaky(jnp.dot(o, cr_ref[...], preferred_element_type=jnp.float32)
               + aggr + cb_ref[...])
    wih = wih_ref[...]
    whh = whh_ref[...]
    bih = bih_ref[...]
    bhh = bhh_ref[...]

    def gate(xv, w, k):
        return lax.dot_general(xv, w[32 * k:32 * (k + 1), :],
                               (((1,), (1,)), ((), ())),
                               preferred_element_type=jnp.float32)

    i_r = gate(m, wih, 0) + bih[0:1, :]
    i_z = gate(m, wih, 1) + bih[1:2, :]
    i_n = gate(m, wih, 2) + bih[2:3, :]
    h_r = gate(o, whh, 0) + bhh[0:1, :]
    h_z = gate(o, whh, 1) + bhh[1:2, :]
    h_n = gate(o, whh, 2) + bhh[2:3, :]
    r = jax.nn.sigmoid(i_r + h_r)
    z = jax.nn.sigmoid(i_z + h_z)
    n = jnp.tanh(i_n + r * h_n)
    h = (1.0 - z) * n + z * o
    h_ref[...] = h
    p_ref[...] = lax.dot_general(h, bond_ref[...], (((1,), (1,)), ((), ())),
                                 preferred_element_type=jnp.float32)


def _pool_body(o_ref, b_ref, res_ref, acc_ref):
    i = pl.program_id(0)

    @pl.when(i == 0)
    def _():
        acc_ref[...] = jnp.zeros((G, NEMB + 1), jnp.float32)

    bb = b_ref[0, 0, :]
    oh = (bb[:, None] == lax.broadcasted_iota(jnp.int32, (R, G), 1)
          ).astype(jnp.float32)
    aug = jnp.concatenate([o_ref[...], jnp.ones((R, 1), jnp.float32)], axis=1)
    acc_ref[...] += lax.dot_general(oh, aug, (((0,), (0,)), ((), ())),
                                    preferred_element_type=jnp.float32)

    @pl.when(i == GRID - 1)
    def _():
        s = acc_ref[...]
        res_ref[...] = s[:, :NEMB] / jnp.maximum(s[:, NEMB:NEMB + 1], 1.0)


def _full(shape):
    return pl.BlockSpec(shape, lambda i: tuple(0 for _ in shape))


_encode = pl.pallas_call(
    _encode_body,
    grid=(GRID,),
    in_specs=[
        pl.BlockSpec((1, 1, R), lambda i: (i, 0, 0)),
        pl.BlockSpec((1, 1, R), lambda i: (i, 0, 0)),
        _full((G, NEMB)),
        _full((NBLK, NEMB)),
        _full((2 * NEMB, NEMB)),
        _full((1, NEMB)),
        _full((NEMB, NEMB)),
        _full((1, NEMB)),
        _full((NSTEM, NEMB)),
    ],
    out_specs=[
        pl.BlockSpec((R, NEMB), lambda i: (i, 0)),
        pl.BlockSpec((R, NSTEM), lambda i: (i, 0)),
    ],
    out_shape=[
        jax.ShapeDtypeStruct((N, NEMB), jnp.float32),
        jax.ShapeDtypeStruct((N, NSTEM), jnp.float32),
    ],
)

_step = pl.pallas_call(
    _step_body,
    grid=(GRID,),
    in_specs=[
        pl.BlockSpec((R, NEMB), lambda i: (i, 0)),
        pl.BlockSpec((NC, R, QCOLS), lambda i: (0, i, 0)),
        _full((NSTEM, NEMB)),
        _full((NEMB, NEMB)),
        _full((1, NEMB)),
        _full((3 * NEMB, NEMB)),
        _full((3 * NEMB, NEMB)),
        _full((3, NEMB)),
        _full((3, NEMB)),
    ],
    out_specs=[
        pl.BlockSpec((R, NEMB), lambda i: (i, 0)),
        pl.BlockSpec((R, NSTEM), lambda i: (i, 0)),
    ],
    out_shape=[
        jax.ShapeDtypeStruct((N, NEMB), jnp.float32),
        jax.ShapeDtypeStruct((N, NSTEM), jnp.float32),
    ],
)

_pool = pl.pallas_call(
    _pool_body,
    grid=(GRID,),
    in_specs=[
        pl.BlockSpec((R, NEMB), lambda i: (i, 0)),
        pl.BlockSpec((1, 1, R), lambda i: (i, 0, 0)),
    ],
    out_specs=pl.BlockSpec((G, NEMB), lambda i: (0, 0)),
    out_shape=jax.ShapeDtypeStruct((G, NEMB), jnp.float32),
    scratch_shapes=[pltpu.VMEM((G, NEMB + 1), jnp.float32)],
)


@functools.partial(
    pl.kernel,
    out_type=jax.ShapeDtypeStruct((NC * QPAD,), jnp.float32),
    mesh=plsc.VectorSubcoreMesh(core_axis_name="c", subcore_axis_name="s",
                                num_cores=NC, num_subcores=NS),
    scratch_types=[
        pltpu.VMEM((CHUNKS, CL), jnp.int32),
        pltpu.VMEM((CHUNKS, CL), jnp.int32),
        pltpu.VMEM((CHUNKS, CL), jnp.int32),
        pltpu.VMEM((CHUNKS, CL), jnp.float32),
        pltpu.VMEM((CL,), jnp.float32),
        pltpu.VMEM((QS,), jnp.float32),
        pltpu.VMEM_SHARED((QPAD,), jnp.float32),
        pltpu.SemaphoreType.DMA,
    ],
)
def _sc_edge(p_hbm, gidx_hbm, sidx_hbm, didx_hbm, zeros_hbm, out_hbm,
             gi_v, si_v, di_v, vals_v, ones_v, stage_v, qsh, sem):
    c = lax.axis_index("c")
    s = lax.axis_index("s")
    w = c * NS + s
    pltpu.sync_copy(gidx_hbm.at[w], gi_v)
    pltpu.sync_copy(sidx_hbm.at[w], si_v)
    pltpu.sync_copy(didx_hbm.at[w], di_v)
    pltpu.sync_copy(zeros_hbm.at[pl.ds(s * QS, QS)], stage_v)
    pltpu.sync_copy(stage_v, qsh.at[pl.ds(s * QS, QS)])
    for i in range(CL // 16):
        ones_v[pl.ds(16 * i, 16)] = jnp.full((16,), 1.0, jnp.float32)
    plsc.subcore_barrier()

    def body(j, carry):
        pltpu.async_copy(p_hbm.at[gi_v.at[j]], vals_v.at[j], sem).wait()
        pltpu.sync_copy(vals_v.at[j], qsh.at[si_v.at[j]], add=True)
        pltpu.sync_copy(ones_v, qsh.at[di_v.at[j]], add=True)
        return carry

    lax.fori_loop(0, CHUNKS, body, 0)
    plsc.subcore_barrier()
    pltpu.sync_copy(qsh.at[pl.ds(s * QS, QS)], stage_v)
    pltpu.sync_copy(stage_v, out_hbm.at[pl.ds(c * QPAD + s * QS, QS)])


def kernel(x, edge_index, edge_attr, batch, vec_data, block_emb, bond_emb,
           W1, b1, W2, b2, conv_root, conv_bias, W_ih, W_hh, b_ih, b_hh):
    x3 = x.astype(jnp.int32).reshape(GRID, 1, R)
    batch3 = batch.astype(jnp.int32).reshape(GRID, 1, R)
    src = edge_index[0].astype(jnp.int32)
    dst = edge_index[1].astype(jnp.int32)
    a0 = edge_attr[:, 0].astype(jnp.int32)
    a1 = edge_attr[:, 1].astype(jnp.int32)

    pad = EPAD - E
    gidx = jnp.concatenate([src * NSTEM + a0, jnp.zeros((pad,), jnp.int32)])
    sidx = jnp.concatenate([dst * QCOLS + a1,
                            jnp.full((pad,), QSIZE + 40, jnp.int32)])
    didx = jnp.concatenate([dst * QCOLS + NSTEM,
                            jnp.full((pad,), QSIZE + 41, jnp.int32)])
    gidx = gidx.reshape(NW, CHUNKS, CL)
    sidx = sidx.reshape(NW, CHUNKS, CL)
    didx = didx.reshape(NW, CHUNKS, CL)
    zeros = jnp.zeros((QPAD,), jnp.float32)

    b1r = b1.reshape(1, NEMB)
    b2r = b2.reshape(1, NEMB)
    cbr = conv_bias.reshape(1, NEMB)
    bihr = b_ih.reshape(3, NEMB)
    bhhr = b_hh.reshape(3, NEMB)

    out, P = _encode(x3, batch3, vec_data, block_emb, W1, b1r, W2, b2r,
                     bond_emb)
    for _ in range(STEPS):
        qraw = _sc_edge(P.reshape(PSIZE), gidx, sidx, didx, zeros)
        q2 = qraw.reshape(NC, QPAD)[:, :QSIZE].reshape(NC, N, QCOLS)
        out, P = _step(out, q2, bond_emb, conv_root, cbr, W_ih, W_hh,
                       bihr, bhhr)
    return _pool(out, batch3)


# speedup vs baseline: 20.2086x; 1.2262x over previous
import functools

import jax
import jax.numpy as jnp
from jax import lax
from jax.experimental import pallas as pl
from jax.experimental.pallas import tpu as pltpu
from jax.experimental.pallas import tpu_sc as plsc

N = 10000
E = 160000
G = 512
NEMB = 32
NSTEM = 20
NBLK = 106
STEPS = 3

R = 1000
GRID = N // R

NC, NS = 2, 16
NW = NC * NS
CL = 128
CHUNKS = 40
EPW = CHUNKS * CL
EPAD = NW * EPW
QCOLS = NSTEM + 1
QSIZE = N * QCOLS
QS = 13128
QPAD = QS * NS
PSIZE = N * NSTEM


def _leaky(t):
    return jnp.where(t >= 0, t, 0.01 * t)


def _encode_body(x_ref, b_ref, vec_ref, blk_ref, w1_ref, b1_ref, w2_ref,
                 b2_ref, bond_ref, out_ref, p_ref):
    xb = x_ref[0, 0, :]
    oh_x = (xb[:, None] == lax.broadcasted_iota(jnp.int32, (R, NBLK), 1)
            ).astype(jnp.float32)
    hx = jnp.dot(oh_x, blk_ref[...], preferred_element_type=jnp.float32)
    bb = b_ref[0, 0, :]
    oh_b = (bb[:, None] == lax.broadcasted_iota(jnp.int32, (R, G), 1)
            ).astype(jnp.float32)
    bv = jnp.dot(oh_b, vec_ref[...], preferred_element_type=jnp.float32)
    cat = jnp.concatenate([hx, bv], axis=1)
    t = _leaky(jnp.dot(cat, w1_ref[...], preferred_element_type=jnp.float32)
               + b1_ref[...])
    out = jnp.dot(t, w2_ref[...], preferred_element_type=jnp.float32) + b2_ref[...]
    out_ref[...] = out
    p_ref[...] = lax.dot_general(out, bond_ref[...], (((1,), (1,)), ((), ())),
                                 preferred_element_type=jnp.float32)


def _gru_tail(o, m, wih_ref, whh_ref, bih_ref, bhh_ref, bond_ref, h_ref,
              p_ref):
    wih = wih_ref[...]
    whh = whh_ref[...]
    bih = bih_ref[...]
    bhh = bhh_ref[...]

    def gate(xv, w, k):
        return lax.dot_general(xv, w[32 * k:32 * (k + 1), :],
                               (((1,), (1,)), ((), ())),
                               preferred_element_type=jnp.float32)

    r = jax.nn.sigmoid(gate(m, wih, 0) + bih[0:1, :]
                       + gate(o, whh, 0) + bhh[0:1, :])
    z = jax.nn.sigmoid(gate(m, wih, 1) + bih[1:2, :]
                       + gate(o, whh, 1) + bhh[1:2, :])
    n = jnp.tanh(gate(m, wih, 2) + bih[2:3, :]
                 + r * (gate(o, whh, 2) + bhh[2:3, :]))
    h = (1.0 - z) * n + z * o
    h_ref[...] = h
    p_ref[...] = lax.dot_general(h, bond_ref[...], (((1,), (1,)), ((), ())),
                                 preferred_element_type=jnp.float32)


def _step_first_body(o_ref, q_ref, bond_ref, cr_ref, cb_ref, wih_ref, whh_ref,
                     bih_ref, bhh_ref, h_ref, p_ref, dinv_ref):
    o = o_ref[...]
    q = q_ref[0] + q_ref[1]
    dinv = 1.0 / jnp.maximum(q[:, NSTEM:NSTEM + 1], 1.0)
    dinv_ref[...] = dinv
    aggr = jnp.dot(q[:, :NSTEM], bond_ref[...],
                   preferred_element_type=jnp.float32) * dinv
    m = _leaky(jnp.dot(o, cr_ref[...], preferred_element_type=jnp.float32)
               + aggr + cb_ref[...])
    _gru_tail(o, m, wih_ref, whh_ref, bih_ref, bhh_ref, bond_ref, h_ref,
              p_ref)


def _step_rest_body(o_ref, q_ref, dinv_ref, bond_ref, cr_ref, cb_ref, wih_ref,
                    whh_ref, bih_ref, bhh_ref, h_ref, p_ref):
    o = o_ref[...]
    q = q_ref[0] + q_ref[1]
    aggr = jnp.dot(q[:, :NSTEM], bond_ref[...],
                   preferred_element_type=jnp.float32) * dinv_ref[...]
    m = _leaky(jnp.dot(o, cr_ref[...], preferred_element_type=jnp.float32)
               + aggr + cb_ref[...])
    _gru_tail(o, m, wih_ref, whh_ref, bih_ref, bhh_ref, bond_ref, h_ref,
              p_ref)


def _pool_body(o_ref, b_ref, res_ref, acc_ref):
    i = pl.program_id(0)

    @pl.when(i == 0)
    def _():
        acc_ref[...] = jnp.zeros((G, NEMB + 1), jnp.float32)

    bb = b_ref[0, 0, :]
    oh = (bb[:, None] == lax.broadcasted_iota(jnp.int32, (R, G), 1)
          ).astype(jnp.float32)
    aug = jnp.concatenate([o_ref[...], jnp.ones((R, 1), jnp.float32)], axis=1)
    acc_ref[...] += lax.dot_general(oh, aug, (((0,), (0,)), ((), ())),
                                    preferred_element_type=jnp.float32)

    @pl.when(i == GRID - 1)
    def _():
        s = acc_ref[...]
        res_ref[...] = s[:, :NEMB] / jnp.maximum(s[:, NEMB:NEMB + 1], 1.0)


def _full(shape):
    return pl.BlockSpec(shape, lambda i: tuple(0 for _ in shape))


_encode = pl.pallas_call(
    _encode_body,
    grid=(GRID,),
    in_specs=[
        pl.BlockSpec((1, 1, R), lambda i: (i, 0, 0)),
        pl.BlockSpec((1, 1, R), lambda i: (i, 0, 0)),
        _full((G, NEMB)),
        _full((NBLK, NEMB)),
        _full((2 * NEMB, NEMB)),
        _full((1, NEMB)),
        _full((NEMB, NEMB)),
        _full((1, NEMB)),
        _full((NSTEM, NEMB)),
    ],
    out_specs=[
        pl.BlockSpec((R, NEMB), lambda i: (i, 0)),
        pl.BlockSpec((R, NSTEM), lambda i: (i, 0)),
    ],
    out_shape=[
        jax.ShapeDtypeStruct((N, NEMB), jnp.float32),
        jax.ShapeDtypeStruct((N, NSTEM), jnp.float32),
    ],
)

_WSPECS = [
    _full((NSTEM, NEMB)),
    _full((NEMB, NEMB)),
    _full((1, NEMB)),
    _full((3 * NEMB, NEMB)),
    _full((3 * NEMB, NEMB)),
    _full((3, NEMB)),
    _full((3, NEMB)),
]

_step_first = pl.pallas_call(
    _step_first_body,
    grid=(GRID,),
    in_specs=[
        pl.BlockSpec((R, NEMB), lambda i: (i, 0)),
        pl.BlockSpec((NC, R, QCOLS), lambda i: (0, i, 0)),
    ] + _WSPECS,
    out_specs=[
        pl.BlockSpec((R, NEMB), lambda i: (i, 0)),
        pl.BlockSpec((R, NSTEM), lambda i: (i, 0)),
        pl.BlockSpec((R, 1), lambda i: (i, 0)),
    ],
    out_shape=[
        jax.ShapeDtypeStruct((N, NEMB), jnp.float32),
        jax.ShapeDtypeStruct((N, NSTEM), jnp.float32),
        jax.ShapeDtypeStruct((N, 1), jnp.float32),
    ],
)

_step_rest = pl.pallas_call(
    _step_rest_body,
    grid=(GRID,),
    in_specs=[
        pl.BlockSpec((R, NEMB), lambda i: (i, 0)),
        pl.BlockSpec((NC, R, QCOLS), lambda i: (0, i, 0)),
        pl.BlockSpec((R, 1), lambda i: (i, 0)),
    ] + _WSPECS,
    out_specs=[
        pl.BlockSpec((R, NEMB), lambda i: (i, 0)),
        pl.BlockSpec((R, NSTEM), lambda i: (i, 0)),
    ],
    out_shape=[
        jax.ShapeDtypeStruct((N, NEMB), jnp.float32),
        jax.ShapeDtypeStruct((N, NSTEM), jnp.float32),
    ],
)

_pool = pl.pallas_call(
    _pool_body,
    grid=(GRID,),
    in_specs=[
        pl.BlockSpec((R, NEMB), lambda i: (i, 0)),
        pl.BlockSpec((1, 1, R), lambda i: (i, 0, 0)),
    ],
    out_specs=pl.BlockSpec((G, NEMB), lambda i: (0, 0)),
    out_shape=jax.ShapeDtypeStruct((G, NEMB), jnp.float32),
    scratch_shapes=[pltpu.VMEM((G, NEMB + 1), jnp.float32)],
)


_DEPTH = 8


def _make_sc_edge(with_deg):
    def body(p_hbm, gidx_hbm, sidx_hbm, didx_hbm, zeros_hbm, out_hbm,
             gi_v, si_v, di_v, vals_v, ones_v, stage_v, qsh, gsem, ssem):
        c = lax.axis_index("c")
        s = lax.axis_index("s")
        w = c * NS + s
        pltpu.sync_copy(gidx_hbm.at[w], gi_v)
        pltpu.sync_copy(sidx_hbm.at[w], si_v)
        if with_deg:
            pltpu.sync_copy(didx_hbm.at[w], di_v)
            for i in range(CL // 16):
                ones_v[pl.ds(16 * i, 16)] = jnp.full((16,), 1.0, jnp.float32)
        pltpu.sync_copy(zeros_hbm.at[pl.ds(s * QS, QS)], stage_v)
        pltpu.sync_copy(stage_v, qsh.at[pl.ds(s * QS, QS)])
        plsc.subcore_barrier()

        for j in range(_DEPTH):
            pltpu.async_copy(p_hbm.at[gi_v.at[j]], vals_v.at[j], gsem.at[j])

        def loop(j, carry):
            slot = lax.rem(j, _DEPTH)
            pltpu.make_async_copy(p_hbm.at[gi_v.at[j]], vals_v.at[j],
                                  gsem.at[slot]).wait()

            @pl.when(j < CHUNKS - _DEPTH)
            def _():
                pltpu.async_copy(p_hbm.at[gi_v.at[j + _DEPTH]],
                                 vals_v.at[j + _DEPTH], gsem.at[slot])

            pltpu.async_copy(vals_v.at[j], qsh.at[si_v.at[j]], ssem,
                             add=True)
            if with_deg:
                pltpu.async_copy(ones_v, qsh.at[di_v.at[j]], ssem, add=True)
            return carry

        lax.fori_loop(0, CHUNKS, loop, 0)

        n_drain = 2 * CHUNKS if with_deg else CHUNKS

        def drain(j, carry):
            pltpu.make_async_copy(vals_v.at[0], qsh.at[si_v.at[0]],
                                  ssem).wait()
            return carry

        lax.fori_loop(0, n_drain, drain, 0)
        plsc.subcore_barrier()
        pltpu.sync_copy(qsh.at[pl.ds(s * QS, QS)], stage_v)
        pltpu.sync_copy(stage_v, out_hbm.at[pl.ds(c * QPAD + s * QS, QS)])

    return pl.kernel(
        body,
        out_type=jax.ShapeDtypeStruct((NC * QPAD,), jnp.float32),
        mesh=plsc.VectorSubcoreMesh(core_axis_name="c", subcore_axis_name="s",
                                    num_cores=NC, num_subcores=NS),
        scratch_types=[
            pltpu.VMEM((CHUNKS, CL), jnp.int32),
            pltpu.VMEM((CHUNKS, CL), jnp.int32),
            pltpu.VMEM((CHUNKS, CL), jnp.int32),
            pltpu.VMEM((CHUNKS, CL), jnp.float32),
            pltpu.VMEM((CL,), jnp.float32),
            pltpu.VMEM((QS,), jnp.float32),
            pltpu.VMEM_SHARED((QPAD,), jnp.float32),
            pltpu.SemaphoreType.DMA((_DEPTH,)),
            pltpu.SemaphoreType.DMA,
        ],
    )


_sc_edge_first = _make_sc_edge(True)
_sc_edge_rest = _make_sc_edge(False)


def kernel(x, edge_index, edge_attr, batch, vec_data, block_emb, bond_emb,
           W1, b1, W2, b2, conv_root, conv_bias, W_ih, W_hh, b_ih, b_hh):
    x3 = x.astype(jnp.int32).reshape(GRID, 1, R)
    batch3 = batch.astype(jnp.int32).reshape(GRID, 1, R)
    src = edge_index[0].astype(jnp.int32)
    dst = edge_index[1].astype(jnp.int32)
    a0 = edge_attr[:, 0].astype(jnp.int32)
    a1 = edge_attr[:, 1].astype(jnp.int32)

    pad = EPAD - E
    gidx = jnp.concatenate([src * NSTEM + a0, jnp.zeros((pad,), jnp.int32)])
    sidx = jnp.concatenate([dst * QCOLS + a1,
                            jnp.full((pad,), QSIZE + 40, jnp.int32)])
    didx = jnp.concatenate([dst * QCOLS + NSTEM,
                            jnp.full((pad,), QSIZE + 41, jnp.int32)])
    gidx = gidx.reshape(NW, CHUNKS, CL)
    sidx = sidx.reshape(NW, CHUNKS, CL)
    didx = didx.reshape(NW, CHUNKS, CL)
    zeros = jnp.zeros((QPAD,), jnp.float32)

    b1r = b1.reshape(1, NEMB)
    b2r = b2.reshape(1, NEMB)
    cbr = conv_bias.reshape(1, NEMB)
    bihr = b_ih.reshape(3, NEMB)
    bhhr = b_hh.reshape(3, NEMB)

    wargs = (bond_emb, conv_root, cbr, W_ih, W_hh, bihr, bhhr)
    out, P = _encode(x3, batch3, vec_data, block_emb, W1, b1r, W2, b2r,
                     bond_emb)
    qraw = _sc_edge_first(P.reshape(PSIZE), gidx, sidx, didx, zeros)
    q2 = qraw.reshape(NC, QPAD)[:, :QSIZE].reshape(NC, N, QCOLS)
    out, P, dinv = _step_first(out, q2, *wargs)
    for _ in range(STEPS - 1):
        qraw = _sc_edge_rest(P.reshape(PSIZE), gidx, sidx, didx, zeros)
        q2 = qraw.reshape(NC, QPAD)[:, :QSIZE].reshape(NC, N, QCOLS)
        out, P = _step_rest(out, q2, dinv, *wargs)
    return _pool(out, batch3)


# speedup vs baseline: 24.2902x; 1.2020x over previous
import jax
import jax.numpy as jnp
from jax import lax
from jax.experimental import pallas as pl
from jax.experimental.pallas import tpu as pltpu
from jax.experimental.pallas import tpu_sc as plsc

N = 10000
E = 160000
G = 512
NEMB = 32
NSTEM = 20
NBLK = 106
STEPS = 3

R = 1000
GRID = N // R

NC, NS = 2, 16
NW = NC * NS
CL = 128
CHUNKS = 40
EPW = CHUNKS * CL
EPAD = NW * EPW
QCOLS = NSTEM + 1
NQ = 10112
QSIZE = NQ * QCOLS
QS = QSIZE // NS
NP = 10016
PFLAT = NP * NSTEM
PTS = PFLAT // NS


def _leaky(t):
    return jnp.where(t >= 0, t, 0.01 * t)


def _encode_body(x_ref, b_ref, vec_ref, blk_ref, w1_ref, b1_ref, w2_ref,
                 b2_ref, bond_ref, out_ref, p_ref):
    xb = x_ref[0, 0, :]
    oh_x = (xb[:, None] == lax.broadcasted_iota(jnp.int32, (R, NBLK), 1)
            ).astype(jnp.float32)
    hx = jnp.dot(oh_x, blk_ref[...], preferred_element_type=jnp.float32)
    bb = b_ref[0, 0, :]
    oh_b = (bb[:, None] == lax.broadcasted_iota(jnp.int32, (R, G), 1)
            ).astype(jnp.float32)
    bv = jnp.dot(oh_b, vec_ref[...], preferred_element_type=jnp.float32)
    cat = jnp.concatenate([hx, bv], axis=1)
    t = _leaky(jnp.dot(cat, w1_ref[...], preferred_element_type=jnp.float32)
               + b1_ref[...])
    out = jnp.dot(t, w2_ref[...], preferred_element_type=jnp.float32) + b2_ref[...]
    out_ref[...] = out
    p_ref[...] = lax.dot_general(out, bond_ref[...], (((1,), (1,)), ((), ())),
                                 preferred_element_type=jnp.float32)


def _gru(o, m, wih_ref, whh_ref, bih_ref, bhh_ref):
    wih = wih_ref[...]
    whh = whh_ref[...]
    bih = bih_ref[...]
    bhh = bhh_ref[...]

    def gate(xv, w, k):
        return lax.dot_general(xv, w[32 * k:32 * (k + 1), :],
                               (((1,), (1,)), ((), ())),
                               preferred_element_type=jnp.float32)

    r = jax.nn.sigmoid(gate(m, wih, 0) + bih[0:1, :]
                       + gate(o, whh, 0) + bhh[0:1, :])
    z = jax.nn.sigmoid(gate(m, wih, 1) + bih[1:2, :]
                       + gate(o, whh, 1) + bhh[1:2, :])
    n = jnp.tanh(gate(m, wih, 2) + bih[2:3, :]
                 + r * (gate(o, whh, 2) + bhh[2:3, :]))
    return (1.0 - z) * n + z * o


def _step_first_body(o_ref, q_ref, bond_ref, cr_ref, cb_ref, wih_ref, whh_ref,
                     bih_ref, bhh_ref, h_ref, p_ref, dinv_ref):
    o = o_ref[...]
    q = q_ref[0] + q_ref[1]
    dinv = 1.0 / jnp.maximum(q[:, NSTEM:NSTEM + 1], 1.0)
    dinv_ref[...] = dinv
    aggr = jnp.dot(q[:, :NSTEM], bond_ref[...],
                   preferred_element_type=jnp.float32) * dinv
    m = _leaky(jnp.dot(o, cr_ref[...], preferred_element_type=jnp.float32)
               + aggr + cb_ref[...])
    h = _gru(o, m, wih_ref, whh_ref, bih_ref, bhh_ref)
    h_ref[...] = h
    p_ref[...] = lax.dot_general(h, bond_ref[...], (((1,), (1,)), ((), ())),
                                 preferred_element_type=jnp.float32)


def _step_rest_body(o_ref, q_ref, dinv_ref, bond_ref, cr_ref, cb_ref, wih_ref,
                    whh_ref, bih_ref, bhh_ref, h_ref, p_ref):
    o = o_ref[...]
    q = q_ref[0] + q_ref[1]
    aggr = jnp.dot(q[:, :NSTEM], bond_ref[...],
                   preferred_element_type=jnp.float32) * dinv_ref[...]
    m = _leaky(jnp.dot(o, cr_ref[...], preferred_element_type=jnp.float32)
               + aggr + cb_ref[...])
    h = _gru(o, m, wih_ref, whh_ref, bih_ref, bhh_ref)
    h_ref[...] = h
    p_ref[...] = lax.dot_general(h, bond_ref[...], (((1,), (1,)), ((), ())),
                                 preferred_element_type=jnp.float32)


def _step_last_body(o_ref, q_ref, dinv_ref, b_ref, bond_ref, cr_ref, cb_ref,
                    wih_ref, whh_ref, bih_ref, bhh_ref, res_ref, acc_ref):
    i = pl.program_id(0)
    o = o_ref[...]
    q = q_ref[0] + q_ref[1]
    aggr = jnp.dot(q[:, :NSTEM], bond_ref[...],
                   preferred_element_type=jnp.float32) * dinv_ref[...]
    m = _leaky(jnp.dot(o, cr_ref[...], preferred_element_type=jnp.float32)
               + aggr + cb_ref[...])
    h = _gru(o, m, wih_ref, whh_ref, bih_ref, bhh_ref)

    @pl.when(i == 0)
    def _():
        acc_ref[...] = jnp.zeros((G, NEMB + 1), jnp.float32)

    bb = b_ref[0, 0, :]
    oh = (bb[:, None] == lax.broadcasted_iota(jnp.int32, (R, G), 1)
          ).astype(jnp.float32)
    aug = jnp.concatenate([h, jnp.ones((R, 1), jnp.float32)], axis=1)
    acc_ref[...] += lax.dot_general(oh, aug, (((0,), (0,)), ((), ())),
                                    preferred_element_type=jnp.float32)

    @pl.when(i == GRID - 1)
    def _():
        sacc = acc_ref[...]
        res_ref[...] = sacc[:, :NEMB] / jnp.maximum(sacc[:, NEMB:NEMB + 1],
                                                    1.0)


def _full(shape):
    return pl.BlockSpec(shape, lambda i: tuple(0 for _ in shape))


_encode = pl.pallas_call(
    _encode_body,
    grid=(GRID,),
    in_specs=[
        pl.BlockSpec((1, 1, R), lambda i: (i, 0, 0)),
        pl.BlockSpec((1, 1, R), lambda i: (i, 0, 0)),
        _full((G, NEMB)),
        _full((NBLK, NEMB)),
        _full((2 * NEMB, NEMB)),
        _full((1, NEMB)),
        _full((NEMB, NEMB)),
        _full((1, NEMB)),
        _full((NSTEM, NEMB)),
    ],
    out_specs=[
        pl.BlockSpec((R, NEMB), lambda i: (i, 0)),
        pl.BlockSpec((R, NSTEM), lambda i: (i, 0)),
    ],
    out_shape=[
        jax.ShapeDtypeStruct((N, NEMB), jnp.float32),
        jax.ShapeDtypeStruct((NP, NSTEM), jnp.float32),
    ],
)

_WSPECS = [
    _full((NSTEM, NEMB)),
    _full((NEMB, NEMB)),
    _full((1, NEMB)),
    _full((3 * NEMB, NEMB)),
    _full((3 * NEMB, NEMB)),
    _full((3, NEMB)),
    _full((3, NEMB)),
]

_QSPEC = pl.BlockSpec((NC, R, QCOLS), lambda i: (0, i, 0))

_step_first = pl.pallas_call(
    _step_first_body,
    grid=(GRID,),
    in_specs=[pl.BlockSpec((R, NEMB), lambda i: (i, 0)), _QSPEC] + _WSPECS,
    out_specs=[
        pl.BlockSpec((R, NEMB), lambda i: (i, 0)),
        pl.BlockSpec((R, NSTEM), lambda i: (i, 0)),
        pl.BlockSpec((R, 1), lambda i: (i, 0)),
    ],
    out_shape=[
        jax.ShapeDtypeStruct((N, NEMB), jnp.float32),
        jax.ShapeDtypeStruct((NP, NSTEM), jnp.float32),
        jax.ShapeDtypeStruct((N, 1), jnp.float32),
    ],
)

_step_rest = pl.pallas_call(
    _step_rest_body,
    grid=(GRID,),
    in_specs=[
        pl.BlockSpec((R, NEMB), lambda i: (i, 0)),
        _QSPEC,
        pl.BlockSpec((R, 1), lambda i: (i, 0)),
    ] + _WSPECS,
    out_specs=[
        pl.BlockSpec((R, NEMB), lambda i: (i, 0)),
        pl.BlockSpec((R, NSTEM), lambda i: (i, 0)),
    ],
    out_shape=[
        jax.ShapeDtypeStruct((N, NEMB), jnp.float32),
        jax.ShapeDtypeStruct((NP, NSTEM), jnp.float32),
    ],
)

_step_last = pl.pallas_call(
    _step_last_body,
    grid=(GRID,),
    in_specs=[
        pl.BlockSpec((R, NEMB), lambda i: (i, 0)),
        _QSPEC,
        pl.BlockSpec((R, 1), lambda i: (i, 0)),
        pl.BlockSpec((1, 1, R), lambda i: (i, 0, 0)),
    ] + _WSPECS,
    out_specs=pl.BlockSpec((G, NEMB), lambda i: (0, 0)),
    out_shape=jax.ShapeDtypeStruct((G, NEMB), jnp.float32),
    scratch_shapes=[pltpu.VMEM((G, NEMB + 1), jnp.float32)],
)


_DEPTH = 8


def _make_sc_edge(with_deg):
    def body(p_hbm, gidx_hbm, sidx_hbm, didx_hbm, zeros_hbm, out_hbm,
             gi_v, si_v, di_v, vals_v, ones_v, stage_v, psh, qsh, gsem, ssem):
        c = lax.axis_index("c")
        s = lax.axis_index("s")
        w = c * NS + s
        pltpu.sync_copy(gidx_hbm.at[w], gi_v)
        pltpu.sync_copy(sidx_hbm.at[w], si_v)
        if with_deg:
            pltpu.sync_copy(didx_hbm.at[w], di_v)
            for i in range(CL // 16):
                ones_v[pl.ds(16 * i, 16)] = jnp.full((16,), 1.0, jnp.float32)
        pltpu.sync_copy(p_hbm.at[pl.ds(s * PTS, PTS)],
                        stage_v.at[pl.ds(0, PTS)])
        pltpu.sync_copy(stage_v.at[pl.ds(0, PTS)], psh.at[pl.ds(s * PTS, PTS)])
        pltpu.sync_copy(zeros_hbm.at[pl.ds(s * QS, QS)], stage_v)
        pltpu.sync_copy(stage_v, qsh.at[pl.ds(s * QS, QS)])
        plsc.subcore_barrier()

        for j in range(_DEPTH):
            pltpu.async_copy(psh.at[gi_v.at[j]], vals_v.at[j], gsem.at[j])

        def loop(j, carry):
            slot = lax.rem(j, _DEPTH)
            pltpu.make_async_copy(psh.at[gi_v.at[j]], vals_v.at[j],
                                  gsem.at[slot]).wait()

            @pl.when(j < CHUNKS - _DEPTH)
            def _():
                pltpu.async_copy(psh.at[gi_v.at[j + _DEPTH]],
                                 vals_v.at[j + _DEPTH], gsem.at[slot])

            pltpu.async_copy(vals_v.at[j], qsh.at[si_v.at[j]], ssem,
                             add=True)
            if with_deg:
                pltpu.async_copy(ones_v, qsh.at[di_v.at[j]], ssem, add=True)
            return carry

        lax.fori_loop(0, CHUNKS, loop, 0)

        n_drain = 2 * CHUNKS if with_deg else CHUNKS

        def drain(j, carry):
            pltpu.make_async_copy(vals_v.at[0], qsh.at[si_v.at[0]],
                                  ssem).wait()
            return carry

        lax.fori_loop(0, n_drain, drain, 0)
        plsc.subcore_barrier()
        pltpu.sync_copy(qsh.at[pl.ds(s * QS, QS)], stage_v)
        pltpu.sync_copy(stage_v, out_hbm.at[pl.ds(c * QSIZE + s * QS, QS)])

    return pl.kernel(
        body,
        out_type=jax.ShapeDtypeStruct((NC * QSIZE,), jnp.float32),
        mesh=plsc.VectorSubcoreMesh(core_axis_name="c", subcore_axis_name="s",
                                    num_cores=NC, num_subcores=NS),
        scratch_types=[
            pltpu.VMEM((CHUNKS, CL), jnp.int32),
            pltpu.VMEM((CHUNKS, CL), jnp.int32),
            pltpu.VMEM((CHUNKS, CL), jnp.int32),
            pltpu.VMEM((CHUNKS, CL), jnp.float32),
            pltpu.VMEM((CL,), jnp.float32),
            pltpu.VMEM((QS,), jnp.float32),
            pltpu.VMEM_SHARED((PFLAT,), jnp.float32),
            pltpu.VMEM_SHARED((QSIZE,), jnp.float32),
            pltpu.SemaphoreType.DMA((_DEPTH,)),
            pltpu.SemaphoreType.DMA,
        ],
    )


_sc_edge_first = _make_sc_edge(True)
_sc_edge_rest = _make_sc_edge(False)


def kernel(x, edge_index, edge_attr, batch, vec_data, block_emb, bond_emb,
           W1, b1, W2, b2, conv_root, conv_bias, W_ih, W_hh, b_ih, b_hh):
    x3 = x.astype(jnp.int32).reshape(GRID, 1, R)
    batch3 = batch.astype(jnp.int32).reshape(GRID, 1, R)
    src = edge_index[0].astype(jnp.int32)
    dst = edge_index[1].astype(jnp.int32)
    a0 = edge_attr[:, 0].astype(jnp.int32)
    a1 = edge_attr[:, 1].astype(jnp.int32)

    pad = EPAD - E
    gidx = jnp.concatenate([src * NSTEM + a0, jnp.zeros((pad,), jnp.int32)])
    sidx = jnp.concatenate([dst * QCOLS + a1,
                            jnp.full((pad,), N * QCOLS, jnp.int32)])
    didx = jnp.concatenate([dst * QCOLS + NSTEM,
                            jnp.full((pad,), N * QCOLS + 1, jnp.int32)])
    gidx = gidx.reshape(NW, CHUNKS, CL)
    sidx = sidx.reshape(NW, CHUNKS, CL)
    didx = didx.reshape(NW, CHUNKS, CL)
    zeros = jnp.zeros((QSIZE,), jnp.float32)

    b1r = b1.reshape(1, NEMB)
    b2r = b2.reshape(1, NEMB)
    cbr = conv_bias.reshape(1, NEMB)
    bihr = b_ih.reshape(3, NEMB)
    bhhr = b_hh.reshape(3, NEMB)

    wargs = (bond_emb, conv_root, cbr, W_ih, W_hh, bihr, bhhr)
    out, P = _encode(x3, batch3, vec_data, block_emb, W1, b1r, W2, b2r,
                     bond_emb)
    qraw = _sc_edge_first(P.reshape(PFLAT), gidx, sidx, didx, zeros)
    out, P, dinv = _step_first(out, qraw.reshape(NC, NQ, QCOLS), *wargs)
    qraw = _sc_edge_rest(P.reshape(PFLAT), gidx, sidx, didx, zeros)
    out, P = _step_rest(out, qraw.reshape(NC, NQ, QCOLS), dinv, *wargs)
    qraw = _sc_edge_rest(P.reshape(PFLAT), gidx, sidx, didx, zeros)
    return _step_last(out, qraw.reshape(NC, NQ, QCOLS), dinv, batch3, *wargs)


# speedup vs baseline: 24.3447x; 1.0022x over previous
import jax
import jax.numpy as jnp
from jax import lax
from jax.experimental import pallas as pl
from jax.experimental.pallas import tpu as pltpu
from jax.experimental.pallas import tpu_sc as plsc

N = 10000
E = 160000
G = 512
NEMB = 32
NSTEM = 20
NBLK = 106
STEPS = 3

R = 1000
GRID = N // R

NC, NS = 2, 16
NW = NC * NS
CL = 128
CHUNKS = 40
EPW = CHUNKS * CL
EPAD = NW * EPW
QCOLS = NSTEM + 1
NQ = 10112
QSIZE = NQ * QCOLS
QS = QSIZE // NS
NP = 10016
PFLAT = NP * NSTEM
PTS = PFLAT // NS


def _leaky(t):
    return jnp.where(t >= 0, t, 0.01 * t)


def _encode_body(x_ref, b_ref, vec_ref, blk_ref, w1_ref, b1_ref, w2_ref,
                 b2_ref, bond_ref, out_ref, p_ref):
    xb = x_ref[0, 0, :]
    oh_x = (xb[:, None] == lax.broadcasted_iota(jnp.int32, (R, NBLK), 1)
            ).astype(jnp.float32)
    hx = jnp.dot(oh_x, blk_ref[...], preferred_element_type=jnp.float32)
    bb = b_ref[0, 0, :]
    oh_b = (bb[:, None] == lax.broadcasted_iota(jnp.int32, (R, G), 1)
            ).astype(jnp.float32)
    bv = jnp.dot(oh_b, vec_ref[...], preferred_element_type=jnp.float32)
    cat = jnp.concatenate([hx, bv], axis=1)
    t = _leaky(jnp.dot(cat, w1_ref[...], preferred_element_type=jnp.float32)
               + b1_ref[...])
    out = jnp.dot(t, w2_ref[...], preferred_element_type=jnp.float32) + b2_ref[...]
    out_ref[...] = out
    p_ref[...] = lax.dot_general(out, bond_ref[...], (((1,), (1,)), ((), ())),
                                 preferred_element_type=jnp.float32)


def _gru(o, m, wih_ref, whh_ref, bih_ref, bhh_ref):
    wih = wih_ref[...]
    whh = whh_ref[...]
    bih = bih_ref[...]
    bhh = bhh_ref[...]

    def gate(xv, w, k):
        return lax.dot_general(xv, w[32 * k:32 * (k + 1), :],
                               (((1,), (1,)), ((), ())),
                               preferred_element_type=jnp.float32)

    r = jax.nn.sigmoid(gate(m, wih, 0) + bih[0:1, :]
                       + gate(o, whh, 0) + bhh[0:1, :])
    z = jax.nn.sigmoid(gate(m, wih, 1) + bih[1:2, :]
                       + gate(o, whh, 1) + bhh[1:2, :])
    n = jnp.tanh(gate(m, wih, 2) + bih[2:3, :]
                 + r * (gate(o, whh, 2) + bhh[2:3, :]))
    return (1.0 - z) * n + z * o


def _step_first_body(o_ref, q_ref, bond_ref, cr_ref, cb_ref, wih_ref, whh_ref,
                     bih_ref, bhh_ref, h_ref, p_ref, dinv_ref):
    o = o_ref[...]
    q = q_ref[0] + q_ref[1]
    dinv = 1.0 / jnp.maximum(q[:, NSTEM:NSTEM + 1], 1.0)
    dinv_ref[...] = dinv
    aggr = jnp.dot(q[:, :NSTEM], bond_ref[...],
                   preferred_element_type=jnp.float32) * dinv
    m = _leaky(jnp.dot(o, cr_ref[...], preferred_element_type=jnp.float32)
               + aggr + cb_ref[...])
    h = _gru(o, m, wih_ref, whh_ref, bih_ref, bhh_ref)
    h_ref[...] = h
    p_ref[...] = lax.dot_general(h, bond_ref[...], (((1,), (1,)), ((), ())),
                                 preferred_element_type=jnp.float32)


def _step_rest_body(o_ref, q_ref, dinv_ref, bond_ref, cr_ref, cb_ref, wih_ref,
                    whh_ref, bih_ref, bhh_ref, h_ref, p_ref):
    o = o_ref[...]
    q = q_ref[0] + q_ref[1]
    aggr = jnp.dot(q[:, :NSTEM], bond_ref[...],
                   preferred_element_type=jnp.float32) * dinv_ref[...]
    m = _leaky(jnp.dot(o, cr_ref[...], preferred_element_type=jnp.float32)
               + aggr + cb_ref[...])
    h = _gru(o, m, wih_ref, whh_ref, bih_ref, bhh_ref)
    h_ref[...] = h
    p_ref[...] = lax.dot_general(h, bond_ref[...], (((1,), (1,)), ((), ())),
                                 preferred_element_type=jnp.float32)


def _step_last_body(o_ref, q_ref, dinv_ref, b_ref, bond_ref, cr_ref, cb_ref,
                    wih_ref, whh_ref, bih_ref, bhh_ref, res_ref, acc_ref):
    i = pl.program_id(0)
    o = o_ref[...]
    q = q_ref[0] + q_ref[1]
    aggr = jnp.dot(q[:, :NSTEM], bond_ref[...],
                   preferred_element_type=jnp.float32) * dinv_ref[...]
    m = _leaky(jnp.dot(o, cr_ref[...], preferred_element_type=jnp.float32)
               + aggr + cb_ref[...])
    h = _gru(o, m, wih_ref, whh_ref, bih_ref, bhh_ref)

    @pl.when(i == 0)
    def _():
        acc_ref[...] = jnp.zeros((G, NEMB + 1), jnp.float32)

    bb = b_ref[0, 0, :]
    oh = (bb[:, None] == lax.broadcasted_iota(jnp.int32, (R, G), 1)
          ).astype(jnp.float32)
    aug = jnp.concatenate([h, jnp.ones((R, 1), jnp.float32)], axis=1)
    acc_ref[...] += lax.dot_general(oh, aug, (((0,), (0,)), ((), ())),
                                    preferred_element_type=jnp.float32)

    @pl.when(i == GRID - 1)
    def _():
        sacc = acc_ref[...]
        res_ref[...] = sacc[:, :NEMB] / jnp.maximum(sacc[:, NEMB:NEMB + 1],
                                                    1.0)


def _full(shape):
    return pl.BlockSpec(shape, lambda i: tuple(0 for _ in shape))


_encode = pl.pallas_call(
    _encode_body,
    grid=(GRID,),
    in_specs=[
        pl.BlockSpec((1, 1, R), lambda i: (i, 0, 0)),
        pl.BlockSpec((1, 1, R), lambda i: (i, 0, 0)),
        _full((G, NEMB)),
        _full((NBLK, NEMB)),
        _full((2 * NEMB, NEMB)),
        _full((1, NEMB)),
        _full((NEMB, NEMB)),
        _full((1, NEMB)),
        _full((NSTEM, NEMB)),
    ],
    out_specs=[
        pl.BlockSpec((R, NEMB), lambda i: (i, 0)),
        pl.BlockSpec((R, NSTEM), lambda i: (i, 0)),
    ],
    out_shape=[
        jax.ShapeDtypeStruct((N, NEMB), jnp.float32),
        jax.ShapeDtypeStruct((NP, NSTEM), jnp.float32),
    ],
)

_WSPECS = [
    _full((NSTEM, NEMB)),
    _full((NEMB, NEMB)),
    _full((1, NEMB)),
    _full((3 * NEMB, NEMB)),
    _full((3 * NEMB, NEMB)),
    _full((3, NEMB)),
    _full((3, NEMB)),
]

_QSPEC = pl.BlockSpec((NC, R, QCOLS), lambda i: (0, i, 0))

_step_first = pl.pallas_call(
    _step_first_body,
    grid=(GRID,),
    in_specs=[pl.BlockSpec((R, NEMB), lambda i: (i, 0)), _QSPEC] + _WSPECS,
    out_specs=[
        pl.BlockSpec((R, NEMB), lambda i: (i, 0)),
        pl.BlockSpec((R, NSTEM), lambda i: (i, 0)),
        pl.BlockSpec((R, 1), lambda i: (i, 0)),
    ],
    out_shape=[
        jax.ShapeDtypeStruct((N, NEMB), jnp.float32),
        jax.ShapeDtypeStruct((NP, NSTEM), jnp.float32),
        jax.ShapeDtypeStruct((N, 1), jnp.float32),
    ],
)

_step_rest = pl.pallas_call(
    _step_rest_body,
    grid=(GRID,),
    in_specs=[
        pl.BlockSpec((R, NEMB), lambda i: (i, 0)),
        _QSPEC,
        pl.BlockSpec((R, 1), lambda i: (i, 0)),
    ] + _WSPECS,
    out_specs=[
        pl.BlockSpec((R, NEMB), lambda i: (i, 0)),
        pl.BlockSpec((R, NSTEM), lambda i: (i, 0)),
    ],
    out_shape=[
        jax.ShapeDtypeStruct((N, NEMB), jnp.float32),
        jax.ShapeDtypeStruct((NP, NSTEM), jnp.float32),
    ],
)

_step_last = pl.pallas_call(
    _step_last_body,
    grid=(GRID,),
    in_specs=[
        pl.BlockSpec((R, NEMB), lambda i: (i, 0)),
        _QSPEC,
        pl.BlockSpec((R, 1), lambda i: (i, 0)),
        pl.BlockSpec((1, 1, R), lambda i: (i, 0, 0)),
    ] + _WSPECS,
    out_specs=pl.BlockSpec((G, NEMB), lambda i: (0, 0)),
    out_shape=jax.ShapeDtypeStruct((G, NEMB), jnp.float32),
    scratch_shapes=[pltpu.VMEM((G, NEMB + 1), jnp.float32)],
)


def _make_sc_edge(with_deg):
    def body(p_hbm, gidx_hbm, sidx_hbm, didx_hbm, zeros_hbm, out_hbm,
             gi_v, si_v, di_v, vals_v, ones_v, stage_v, zst_v, psh, qsh,
             isem, gsem, ssem):
        c = lax.axis_index("c")
        s = lax.axis_index("s")
        w = c * NS + s
        pltpu.async_copy(gidx_hbm.at[w], gi_v, isem.at[0])
        pltpu.async_copy(sidx_hbm.at[w], si_v, isem.at[1])
        if with_deg:
            pltpu.async_copy(didx_hbm.at[w], di_v, isem.at[2])
            for i in range(CL // 16):
                ones_v[pl.ds(16 * i, 16)] = jnp.full((16,), 1.0, jnp.float32)
        pltpu.async_copy(p_hbm.at[pl.ds(s * PTS, PTS)],
                         stage_v.at[pl.ds(0, PTS)], isem.at[3])
        pltpu.async_copy(zeros_hbm.at[pl.ds(s * QS, QS)], zst_v, isem.at[4])
        pltpu.make_async_copy(p_hbm.at[pl.ds(s * PTS, PTS)],
                              stage_v.at[pl.ds(0, PTS)], isem.at[3]).wait()
        pltpu.sync_copy(stage_v.at[pl.ds(0, PTS)], psh.at[pl.ds(s * PTS, PTS)])
        pltpu.make_async_copy(zeros_hbm.at[pl.ds(s * QS, QS)], zst_v,
                              isem.at[4]).wait()
        pltpu.sync_copy(zst_v, qsh.at[pl.ds(s * QS, QS)])
        pltpu.make_async_copy(gidx_hbm.at[w], gi_v, isem.at[0]).wait()
        if with_deg:
            pltpu.make_async_copy(didx_hbm.at[w], di_v, isem.at[2]).wait()
        plsc.subcore_barrier()

        def gfire(j, carry):
            pltpu.async_copy(psh.at[gi_v.at[j]], vals_v.at[j], gsem)
            return carry

        lax.fori_loop(0, CHUNKS, gfire, 0)
        pltpu.make_async_copy(sidx_hbm.at[w], si_v, isem.at[1]).wait()
        if with_deg:
            def dscat(j, carry):
                pltpu.async_copy(ones_v, qsh.at[di_v.at[j]], ssem, add=True)
                return carry

            lax.fori_loop(0, CHUNKS, dscat, 0)

        def gdrain(j, carry):
            pltpu.make_async_copy(psh.at[gi_v.at[0]], vals_v.at[0],
                                  gsem).wait()
            return carry

        lax.fori_loop(0, CHUNKS, gdrain, 0)

        def sfire(j, carry):
            pltpu.async_copy(vals_v.at[j], qsh.at[si_v.at[j]], ssem, add=True)
            return carry

        lax.fori_loop(0, CHUNKS, sfire, 0)

        n_drain = 2 * CHUNKS if with_deg else CHUNKS

        def sdrain(j, carry):
            pltpu.make_async_copy(vals_v.at[0], qsh.at[si_v.at[0]],
                                  ssem).wait()
            return carry

        lax.fori_loop(0, n_drain, sdrain, 0)
        plsc.subcore_barrier()
        pltpu.sync_copy(qsh.at[pl.ds(s * QS, QS)], stage_v)
        pltpu.sync_copy(stage_v, out_hbm.at[pl.ds(c * QSIZE + s * QS, QS)])

    return pl.kernel(
        body,
        out_type=jax.ShapeDtypeStruct((NC * QSIZE,), jnp.float32),
        mesh=plsc.VectorSubcoreMesh(core_axis_name="c", subcore_axis_name="s",
                                    num_cores=NC, num_subcores=NS),
        scratch_types=[
            pltpu.VMEM((CHUNKS, CL), jnp.int32),
            pltpu.VMEM((CHUNKS, CL), jnp.int32),
            pltpu.VMEM((CHUNKS, CL), jnp.int32),
            pltpu.VMEM((CHUNKS, CL), jnp.float32),
            pltpu.VMEM((CL,), jnp.float32),
            pltpu.VMEM((QS,), jnp.float32),
            pltpu.VMEM((QS,), jnp.float32),
            pltpu.VMEM_SHARED((PFLAT,), jnp.float32),
            pltpu.VMEM_SHARED((QSIZE,), jnp.float32),
            pltpu.SemaphoreType.DMA((5,)),
            pltpu.SemaphoreType.DMA,
            pltpu.SemaphoreType.DMA,
        ],
    )


_sc_edge_first = _make_sc_edge(True)
_sc_edge_rest = _make_sc_edge(False)


def kernel(x, edge_index, edge_attr, batch, vec_data, block_emb, bond_emb,
           W1, b1, W2, b2, conv_root, conv_bias, W_ih, W_hh, b_ih, b_hh):
    x3 = x.astype(jnp.int32).reshape(GRID, 1, R)
    batch3 = batch.astype(jnp.int32).reshape(GRID, 1, R)
    src = edge_index[0].astype(jnp.int32)
    dst = edge_index[1].astype(jnp.int32)
    a0 = edge_attr[:, 0].astype(jnp.int32)
    a1 = edge_attr[:, 1].astype(jnp.int32)

    pad = EPAD - E
    gidx = jnp.concatenate([src * NSTEM + a0, jnp.zeros((pad,), jnp.int32)])
    sidx = jnp.concatenate([dst * QCOLS + a1,
                            jnp.full((pad,), N * QCOLS, jnp.int32)])
    didx = jnp.concatenate([dst * QCOLS + NSTEM,
                            jnp.full((pad,), N * QCOLS + 1, jnp.int32)])
    gidx = gidx.reshape(NW, CHUNKS, CL)
    sidx = sidx.reshape(NW, CHUNKS, CL)
    didx = didx.reshape(NW, CHUNKS, CL)
    zeros = jnp.zeros((QSIZE,), jnp.float32)

    b1r = b1.reshape(1, NEMB)
    b2r = b2.reshape(1, NEMB)
    cbr = conv_bias.reshape(1, NEMB)
    bihr = b_ih.reshape(3, NEMB)
    bhhr = b_hh.reshape(3, NEMB)

    wargs = (bond_emb, conv_root, cbr, W_ih, W_hh, bihr, bhhr)
    out, P = _encode(x3, batch3, vec_data, block_emb, W1, b1r, W2, b2r,
                     bond_emb)
    qraw = _sc_edge_first(P.reshape(PFLAT), gidx, sidx, didx, zeros)
    out, P, dinv = _step_first(out, qraw.reshape(NC, NQ, QCOLS), *wargs)
    qraw = _sc_edge_rest(P.reshape(PFLAT), gidx, sidx, didx, zeros)
    out, P = _step_rest(out, qraw.reshape(NC, NQ, QCOLS), dinv, *wargs)
    qraw = _sc_edge_rest(P.reshape(PFLAT), gidx, sidx, didx, zeros)
    return _step_last(out, qraw.reshape(NC, NQ, QCOLS), dinv, batch3, *wargs)


# speedup vs baseline: 26.3585x; 1.0827x over previous
import jax
import jax.numpy as jnp
from jax import lax
from jax.experimental import pallas as pl
from jax.experimental.pallas import tpu as pltpu
from jax.experimental.pallas import tpu_sc as plsc

N = 10000
E = 160000
G = 512
NEMB = 32
NSTEM = 20
NBLK = 106
STEPS = 3

R = 2000
GRID = N // R

NC, NS = 2, 16
NW = NC * NS
CL = 128
CHUNKS = 40
EPW = CHUNKS * CL
EPAD = NW * EPW
QCOLS = NSTEM + 1
NQ = 10112
QSIZE = NQ * QCOLS
QS = QSIZE // NS
NP = 10016
PFLAT = NP * NSTEM
PTS = PFLAT // NS


def _leaky(t):
    return jnp.where(t >= 0, t, 0.01 * t)


def _encode_body(x_ref, b_ref, vec_ref, blk_ref, w1_ref, b1_ref, w2_ref,
                 b2_ref, bond_ref, out_ref, p_ref):
    xb = x_ref[0, 0, :]
    oh_x = (xb[:, None] == lax.broadcasted_iota(jnp.int32, (R, NBLK), 1)
            ).astype(jnp.float32)
    hx = jnp.dot(oh_x, blk_ref[...], preferred_element_type=jnp.float32)
    bb = b_ref[0, 0, :]
    oh_b = (bb[:, None] == lax.broadcasted_iota(jnp.int32, (R, G), 1)
            ).astype(jnp.float32)
    bv = jnp.dot(oh_b, vec_ref[...], preferred_element_type=jnp.float32)
    cat = jnp.concatenate([hx, bv], axis=1)
    t = _leaky(jnp.dot(cat, w1_ref[...], preferred_element_type=jnp.float32)
               + b1_ref[...])
    out = jnp.dot(t, w2_ref[...], preferred_element_type=jnp.float32) + b2_ref[...]
    out_ref[...] = out
    p_ref[...] = lax.dot_general(out, bond_ref[...], (((1,), (1,)), ((), ())),
                                 preferred_element_type=jnp.float32)


def _gru(o, m, wih_ref, whh_ref, bih_ref, bhh_ref):
    wih = wih_ref[...]
    whh = whh_ref[...]
    bih = bih_ref[...]
    bhh = bhh_ref[...]

    def gate(xv, w, k):
        return lax.dot_general(xv, w[32 * k:32 * (k + 1), :],
                               (((1,), (1,)), ((), ())),
                               preferred_element_type=jnp.float32)

    r = jax.nn.sigmoid(gate(m, wih, 0) + bih[0:1, :]
                       + gate(o, whh, 0) + bhh[0:1, :])
    z = jax.nn.sigmoid(gate(m, wih, 1) + bih[1:2, :]
                       + gate(o, whh, 1) + bhh[1:2, :])
    n = jnp.tanh(gate(m, wih, 2) + bih[2:3, :]
                 + r * (gate(o, whh, 2) + bhh[2:3, :]))
    return (1.0 - z) * n + z * o


def _step_first_body(o_ref, q_ref, bond_ref, cr_ref, cb_ref, wih_ref, whh_ref,
                     bih_ref, bhh_ref, h_ref, p_ref, dinv_ref):
    o = o_ref[...]
    q = q_ref[0] + q_ref[1]
    dinv = 1.0 / jnp.maximum(q[:, NSTEM:NSTEM + 1], 1.0)
    dinv_ref[...] = dinv
    aggr = jnp.dot(q[:, :NSTEM], bond_ref[...],
                   preferred_element_type=jnp.float32) * dinv
    m = _leaky(jnp.dot(o, cr_ref[...], preferred_element_type=jnp.float32)
               + aggr + cb_ref[...])
    h = _gru(o, m, wih_ref, whh_ref, bih_ref, bhh_ref)
    h_ref[...] = h
    p_ref[...] = lax.dot_general(h, bond_ref[...], (((1,), (1,)), ((), ())),
                                 preferred_element_type=jnp.float32)


def _step_rest_body(o_ref, q_ref, dinv_ref, bond_ref, cr_ref, cb_ref, wih_ref,
                    whh_ref, bih_ref, bhh_ref, h_ref, p_ref):
    o = o_ref[...]
    q = q_ref[0] + q_ref[1]
    aggr = jnp.dot(q[:, :NSTEM], bond_ref[...],
                   preferred_element_type=jnp.float32) * dinv_ref[...]
    m = _leaky(jnp.dot(o, cr_ref[...], preferred_element_type=jnp.float32)
               + aggr + cb_ref[...])
    h = _gru(o, m, wih_ref, whh_ref, bih_ref, bhh_ref)
    h_ref[...] = h
    p_ref[...] = lax.dot_general(h, bond_ref[...], (((1,), (1,)), ((), ())),
                                 preferred_element_type=jnp.float32)


def _step_last_body(o_ref, q_ref, dinv_ref, b_ref, bond_ref, cr_ref, cb_ref,
                    wih_ref, whh_ref, bih_ref, bhh_ref, res_ref, acc_ref):
    i = pl.program_id(0)
    o = o_ref[...]
    q = q_ref[0] + q_ref[1]
    aggr = jnp.dot(q[:, :NSTEM], bond_ref[...],
                   preferred_element_type=jnp.float32) * dinv_ref[...]
    m = _leaky(jnp.dot(o, cr_ref[...], preferred_element_type=jnp.float32)
               + aggr + cb_ref[...])
    h = _gru(o, m, wih_ref, whh_ref, bih_ref, bhh_ref)

    @pl.when(i == 0)
    def _():
        acc_ref[...] = jnp.zeros((G, NEMB + 1), jnp.float32)

    bb = b_ref[0, 0, :]
    oh = (bb[:, None] == lax.broadcasted_iota(jnp.int32, (R, G), 1)
          ).astype(jnp.float32)
    aug = jnp.concatenate([h, jnp.ones((R, 1), jnp.float32)], axis=1)
    acc_ref[...] += lax.dot_general(oh, aug, (((0,), (0,)), ((), ())),
                                    preferred_element_type=jnp.float32)

    @pl.when(i == GRID - 1)
    def _():
        sacc = acc_ref[...]
        res_ref[...] = sacc[:, :NEMB] / jnp.maximum(sacc[:, NEMB:NEMB + 1],
                                                    1.0)


def _full(shape):
    return pl.BlockSpec(shape, lambda i: tuple(0 for _ in shape))


_encode = pl.pallas_call(
    _encode_body,
    grid=(GRID,),
    in_specs=[
        pl.BlockSpec((1, 1, R), lambda i: (i, 0, 0)),
        pl.BlockSpec((1, 1, R), lambda i: (i, 0, 0)),
        _full((G, NEMB)),
        _full((NBLK, NEMB)),
        _full((2 * NEMB, NEMB)),
        _full((1, NEMB)),
        _full((NEMB, NEMB)),
        _full((1, NEMB)),
        _full((NSTEM, NEMB)),
    ],
    out_specs=[
        pl.BlockSpec((R, NEMB), lambda i: (i, 0)),
        pl.BlockSpec((R, NSTEM), lambda i: (i, 0)),
    ],
    out_shape=[
        jax.ShapeDtypeStruct((N, NEMB), jnp.float32),
        jax.ShapeDtypeStruct((NP, NSTEM), jnp.float32),
    ],
)

_WSPECS = [
    _full((NSTEM, NEMB)),
    _full((NEMB, NEMB)),
    _full((1, NEMB)),
    _full((3 * NEMB, NEMB)),
    _full((3 * NEMB, NEMB)),
    _full((3, NEMB)),
    _full((3, NEMB)),
]

_QSPEC = pl.BlockSpec((NC, R, QCOLS), lambda i: (0, i, 0))

_step_first = pl.pallas_call(
    _step_first_body,
    grid=(GRID,),
    in_specs=[pl.BlockSpec((R, NEMB), lambda i: (i, 0)), _QSPEC] + _WSPECS,
    out_specs=[
        pl.BlockSpec((R, NEMB), lambda i: (i, 0)),
        pl.BlockSpec((R, NSTEM), lambda i: (i, 0)),
        pl.BlockSpec((R, 1), lambda i: (i, 0)),
    ],
    out_shape=[
        jax.ShapeDtypeStruct((N, NEMB), jnp.float32),
        jax.ShapeDtypeStruct((NP, NSTEM), jnp.float32),
        jax.ShapeDtypeStruct((N, 1), jnp.float32),
    ],
)

_step_rest = pl.pallas_call(
    _step_rest_body,
    grid=(GRID,),
    in_specs=[
        pl.BlockSpec((R, NEMB), lambda i: (i, 0)),
        _QSPEC,
        pl.BlockSpec((R, 1), lambda i: (i, 0)),
    ] + _WSPECS,
    out_specs=[
        pl.BlockSpec((R, NEMB), lambda i: (i, 0)),
        pl.BlockSpec((R, NSTEM), lambda i: (i, 0)),
    ],
    out_shape=[
        jax.ShapeDtypeStruct((N, NEMB), jnp.float32),
        jax.ShapeDtypeStruct((NP, NSTEM), jnp.float32),
    ],
)

_step_last = pl.pallas_call(
    _step_last_body,
    grid=(GRID,),
    in_specs=[
        pl.BlockSpec((R, NEMB), lambda i: (i, 0)),
        _QSPEC,
        pl.BlockSpec((R, 1), lambda i: (i, 0)),
        pl.BlockSpec((1, 1, R), lambda i: (i, 0, 0)),
    ] + _WSPECS,
    out_specs=pl.BlockSpec((G, NEMB), lambda i: (0, 0)),
    out_shape=jax.ShapeDtypeStruct((G, NEMB), jnp.float32),
    scratch_shapes=[pltpu.VMEM((G, NEMB + 1), jnp.float32)],
)


def _make_sc_edge(with_deg):
    def body(p_hbm, gidx_hbm, sidx_hbm, didx_hbm, zeros_hbm, out_hbm,
             gi_v, si_v, di_v, vals_v, ones_v, stage_v, zst_v, psh, qsh,
             isem, gsem, ssem):
        c = lax.axis_index("c")
        s = lax.axis_index("s")
        w = c * NS + s
        pltpu.async_copy(gidx_hbm.at[w], gi_v, isem.at[0])
        pltpu.async_copy(sidx_hbm.at[w], si_v, isem.at[1])
        if with_deg:
            pltpu.async_copy(didx_hbm.at[w], di_v, isem.at[2])
            for i in range(CL // 16):
                ones_v[pl.ds(16 * i, 16)] = jnp.full((16,), 1.0, jnp.float32)
        pltpu.async_copy(p_hbm.at[pl.ds(s * PTS, PTS)],
                         stage_v.at[pl.ds(0, PTS)], isem.at[3])
        pltpu.async_copy(zeros_hbm.at[pl.ds(s * QS, QS)], zst_v, isem.at[4])
        pltpu.make_async_copy(p_hbm.at[pl.ds(s * PTS, PTS)],
                              stage_v.at[pl.ds(0, PTS)], isem.at[3]).wait()
        pltpu.sync_copy(stage_v.at[pl.ds(0, PTS)], psh.at[pl.ds(s * PTS, PTS)])
        pltpu.make_async_copy(zeros_hbm.at[pl.ds(s * QS, QS)], zst_v,
                              isem.at[4]).wait()
        pltpu.sync_copy(zst_v, qsh.at[pl.ds(s * QS, QS)])
        pltpu.make_async_copy(gidx_hbm.at[w], gi_v, isem.at[0]).wait()
        if with_deg:
            pltpu.make_async_copy(didx_hbm.at[w], di_v, isem.at[2]).wait()
        plsc.subcore_barrier()

        def gfire(j, carry):
            pltpu.async_copy(psh.at[gi_v.at[j]], vals_v.at[j], gsem)
            return carry

        lax.fori_loop(0, CHUNKS, gfire, 0)
        pltpu.make_async_copy(sidx_hbm.at[w], si_v, isem.at[1]).wait()
        if with_deg:
            def dscat(j, carry):
                pltpu.async_copy(ones_v, qsh.at[di_v.at[j]], ssem, add=True)
                return carry

            lax.fori_loop(0, CHUNKS, dscat, 0)

        def gdrain(j, carry):
            pltpu.make_async_copy(psh.at[gi_v.at[0]], vals_v.at[0],
                                  gsem).wait()
            return carry

        lax.fori_loop(0, CHUNKS, gdrain, 0)

        def sfire(j, carry):
            pltpu.async_copy(vals_v.at[j], qsh.at[si_v.at[j]], ssem, add=True)
            return carry

        lax.fori_loop(0, CHUNKS, sfire, 0)

        n_drain = 2 * CHUNKS if with_deg else CHUNKS

        def sdrain(j, carry):
            pltpu.make_async_copy(vals_v.at[0], qsh.at[si_v.at[0]],
                                  ssem).wait()
            return carry

        lax.fori_loop(0, n_drain, sdrain, 0)
        plsc.subcore_barrier()
        pltpu.sync_copy(qsh.at[pl.ds(s * QS, QS)], stage_v)
        pltpu.sync_copy(stage_v, out_hbm.at[pl.ds(c * QSIZE + s * QS, QS)])

    return pl.kernel(
        body,
        out_type=jax.ShapeDtypeStruct((NC * QSIZE,), jnp.float32),
        mesh=plsc.VectorSubcoreMesh(core_axis_name="c", subcore_axis_name="s",
                                    num_cores=NC, num_subcores=NS),
        scratch_types=[
            pltpu.VMEM((CHUNKS, CL), jnp.int32),
            pltpu.VMEM((CHUNKS, CL), jnp.int32),
            pltpu.VMEM((CHUNKS, CL), jnp.int32),
            pltpu.VMEM((CHUNKS, CL), jnp.float32),
            pltpu.VMEM((CL,), jnp.float32),
            pltpu.VMEM((QS,), jnp.float32),
            pltpu.VMEM((QS,), jnp.float32),
            pltpu.VMEM_SHARED((PFLAT,), jnp.float32),
            pltpu.VMEM_SHARED((QSIZE,), jnp.float32),
            pltpu.SemaphoreType.DMA((5,)),
            pltpu.SemaphoreType.DMA,
            pltpu.SemaphoreType.DMA,
        ],
    )


_sc_edge_first = _make_sc_edge(True)
_sc_edge_rest = _make_sc_edge(False)


def kernel(x, edge_index, edge_attr, batch, vec_data, block_emb, bond_emb,
           W1, b1, W2, b2, conv_root, conv_bias, W_ih, W_hh, b_ih, b_hh):
    x3 = x.astype(jnp.int32).reshape(GRID, 1, R)
    batch3 = batch.astype(jnp.int32).reshape(GRID, 1, R)
    src = edge_index[0].astype(jnp.int32)
    dst = edge_index[1].astype(jnp.int32)
    a0 = edge_attr[:, 0].astype(jnp.int32)
    a1 = edge_attr[:, 1].astype(jnp.int32)

    pad = EPAD - E
    gidx = jnp.concatenate([src * NSTEM + a0, jnp.zeros((pad,), jnp.int32)])
    sidx = jnp.concatenate([dst * QCOLS + a1,
                            jnp.full((pad,), N * QCOLS, jnp.int32)])
    didx = jnp.concatenate([dst * QCOLS + NSTEM,
                            jnp.full((pad,), N * QCOLS + 1, jnp.int32)])
    gidx = gidx.reshape(NW, CHUNKS, CL)
    sidx = sidx.reshape(NW, CHUNKS, CL)
    didx = didx.reshape(NW, CHUNKS, CL)
    zeros = jnp.zeros((QSIZE,), jnp.float32)

    b1r = b1.reshape(1, NEMB)
    b2r = b2.reshape(1, NEMB)
    cbr = conv_bias.reshape(1, NEMB)
    bihr = b_ih.reshape(3, NEMB)
    bhhr = b_hh.reshape(3, NEMB)

    wargs = (bond_emb, conv_root, cbr, W_ih, W_hh, bihr, bhhr)
    out, P = _encode(x3, batch3, vec_data, block_emb, W1, b1r, W2, b2r,
                     bond_emb)
    qraw = _sc_edge_first(P.reshape(PFLAT), gidx, sidx, didx, zeros)
    out, P, dinv = _step_first(out, qraw.reshape(NC, NQ, QCOLS), *wargs)
    qraw = _sc_edge_rest(P.reshape(PFLAT), gidx, sidx, didx, zeros)
    out, P = _step_rest(out, qraw.reshape(NC, NQ, QCOLS), dinv, *wargs)
    qraw = _sc_edge_rest(P.reshape(PFLAT), gidx, sidx, didx, zeros)
    return _step_last(out, qraw.reshape(NC, NQ, QCOLS), dinv, batch3, *wargs)
